# baseline (speedup 1.0000x reference)
import functools

import jax
import jax.numpy as jnp
from jax import lax
from jax.experimental import pallas as pl
from jax.experimental.pallas import tpu as pltpu
from jax.experimental.pallas import tpu_sc as plsc

VOCAB_SIZE = 100000
N_TOKENS = 512
LANE = 128
EDGE_START = (VOCAB_SIZE // LANE) * LANE
NUM_CORES = 2
NUM_SUBCORES = 16
NUM_WORKERS = NUM_CORES * NUM_SUBCORES
ROWS_PER_WORKER = N_TOKENS // NUM_WORKERS
BV = 4096


def _tc_fill_kernel(ids_ref, out_ref):
    j = pl.program_id(1)
    col0 = j * BV
    rows, bv = out_ref.shape
    idx = (ids_ref[...] + 1) % VOCAB_SIZE
    idx_eff = jnp.where(idx >= EDGE_START, idx, -1)
    iota = jax.lax.broadcasted_iota(jnp.int32, (rows, bv), 1) + col0
    out_ref[...] = jnp.where(iota == idx_eff, 20.0, 0.0).astype(jnp.float32)


def _tc_fill(ids):
    n = ids.shape[0]
    nr = n // 4
    grid = pl.cdiv(VOCAB_SIZE, BV)
    return pl.pallas_call(
        _tc_fill_kernel,
        grid=(4, grid),
        in_specs=[pl.BlockSpec((nr, 1), lambda i, j: (i, 0))],
        out_specs=pl.BlockSpec((nr, BV), lambda i, j: (i, j)),
        out_shape=jax.ShapeDtypeStruct((n, VOCAB_SIZE), jnp.float32),
    )(ids.reshape(n, 1))


_mesh = plsc.VectorSubcoreMesh(
    core_axis_name="c",
    subcore_axis_name="s",
    num_cores=NUM_CORES,
    num_subcores=NUM_SUBCORES,
)


@functools.partial(
    pl.kernel,
    mesh=_mesh,
    scratch_types=[
        pltpu.VMEM((ROWS_PER_WORKER,), jnp.int32),
        pltpu.VMEM((8, LANE), jnp.float32),
        pltpu.VMEM((8, LANE), jnp.float32),
        pltpu.SemaphoreType.DMA,
        pltpu.SemaphoreType.DMA,
    ],
)
def _sc_patch(ids_hbm, out_ref, ids_v, pb0, pb1, sem0, sem1):
    wid = lax.axis_index("s") * NUM_CORES + lax.axis_index("c")
    base = wid * ROWS_PER_WORKER

    pltpu.sync_copy(ids_hbm.at[pl.ds(base, ROWS_PER_WORKER)], ids_v)
    idx_vec = (ids_v[...] + 1) % VOCAB_SIZE

    lane = lax.iota(jnp.int32, 16)
    zeros16 = jnp.zeros((16,), jnp.float32)

    pbs = [pb0, pb1]
    sems = [sem0, sem1]
    last_desc = [None, None]

    for g in range(2):
        rows0 = base + g * 8

        for hb in range(2):
            if last_desc[hb] is not None:
                last_desc[hb].wait()
                last_desc[hb] = None
            for rr in range(8):
                for k in range(LANE // 16):
                    pbs[hb][rr, pl.ds(k * 16, 16)] = zeros16

        idx_s = [idx_vec[g * 8 + r2] for r2 in range(8)]
        ct_s = [ix // LANE for ix in idx_s]
        edge_s = [ix >= EDGE_START for ix in idx_s]
        b16_s = [((ix % LANE) // 16) * 16 for ix in idx_s]
        o16_s = [ix % 16 for ix in idx_s]
        ct_eff_s = [jnp.where(edge_s[r], 0, ct_s[r]) for r in range(8)]

        for r in range(8):
            hb = r % 2
            if last_desc[hb] is not None:
                last_desc[hb].wait()
            ct = ct_eff_s[r]
            for r2 in range(8):
                contrib = jnp.where(
                    (ct_s[r2] == ct) & jnp.logical_not(edge_s[r2]),
                    jnp.float32(20.0), jnp.float32(0.0))
                pbs[hb][r2, pl.ds(b16_s[r2], 16)] = jnp.where(
                    lane == o16_s[r2], contrib, jnp.float32(0.0))
            d = pltpu.make_async_copy(
                pbs[hb],
                out_ref.at[pl.ds(rows0, 8), pl.ds(ct * LANE, LANE)],
                sems[hb])
            d.start()
            last_desc[hb] = d

    last_desc[0].wait()
    last_desc[1].wait()


def kernel(input_ids):
    B, S = input_ids.shape
    ids = input_ids.reshape(-1).astype(jnp.int32)
    dense = _tc_fill(ids)
    ref = jax.new_ref(dense)
    _sc_patch(ids, ref)
    return ref[...].reshape(B, S, VOCAB_SIZE)


# speedup vs baseline: 1.2119x; 1.2119x over previous
import functools

import jax
import jax.numpy as jnp
from jax import lax
from jax.experimental import pallas as pl
from jax.experimental.pallas import tpu as pltpu
from jax.experimental.pallas import tpu_sc as plsc

VOCAB_SIZE = 100000
N_TOKENS = 512
LANE = 128
EDGE_START = (VOCAB_SIZE // LANE) * LANE
NUM_CORES = 2
NUM_SUBCORES = 16
NUM_WORKERS = NUM_CORES * NUM_SUBCORES
ROWS_PER_WORKER = N_TOKENS // NUM_WORKERS
BV = 4096


def _tc_fill_kernel(ids_ref, out_ref):
    j = pl.program_id(1)
    col0 = j * BV
    rows, bv = out_ref.shape
    idx = (ids_ref[...] + 1) % VOCAB_SIZE
    idx_eff = jnp.where(idx >= EDGE_START, idx, -1)
    iota = jax.lax.broadcasted_iota(jnp.int32, (rows, bv), 1) + col0
    out_ref[...] = jnp.where(iota == idx_eff, 20.0, 0.0).astype(jnp.float32)


def _tc_fill(ids):
    n = ids.shape[0]
    nr = n // 2
    grid = pl.cdiv(VOCAB_SIZE, BV)
    return pl.pallas_call(
        _tc_fill_kernel,
        grid=(2, grid),
        in_specs=[pl.BlockSpec((nr, 1), lambda i, j: (i, 0))],
        out_specs=pl.BlockSpec((nr, BV), lambda i, j: (i, j)),
        out_shape=jax.ShapeDtypeStruct((n, VOCAB_SIZE), jnp.float32),
    )(ids.reshape(n, 1))


_mesh = plsc.VectorSubcoreMesh(
    core_axis_name="c",
    subcore_axis_name="s",
    num_cores=NUM_CORES,
    num_subcores=NUM_SUBCORES,
)


@functools.partial(
    pl.kernel,
    mesh=_mesh,
    scratch_types=[
        pltpu.VMEM((ROWS_PER_WORKER,), jnp.int32),
        pltpu.VMEM((8, LANE), jnp.float32),
        pltpu.VMEM((8, LANE), jnp.float32),
        pltpu.SemaphoreType.DMA,
        pltpu.SemaphoreType.DMA,
    ],
)
def _sc_patch(ids_hbm, out_ref, ids_v, pb0, pb1, sem0, sem1):
    wid = lax.axis_index("s") * NUM_CORES + lax.axis_index("c")
    base = wid * ROWS_PER_WORKER

    pltpu.sync_copy(ids_hbm.at[pl.ds(base, ROWS_PER_WORKER)], ids_v)
    idx_vec = (ids_v[...] + 1) % VOCAB_SIZE

    lane = lax.iota(jnp.int32, 16)
    zeros16 = jnp.zeros((16,), jnp.float32)

    pbs = [pb0, pb1]
    sems = [sem0, sem1]
    last_desc = [None, None]

    for g in range(2):
        rows0 = base + g * 8

        for hb in range(2):
            if last_desc[hb] is not None:
                last_desc[hb].wait()
                last_desc[hb] = None
            for rr in range(8):
                for k in range(LANE // 16):
                    pbs[hb][rr, pl.ds(k * 16, 16)] = zeros16

        idx_s = [idx_vec[g * 8 + r2] for r2 in range(8)]
        ct_s = [ix // LANE for ix in idx_s]
        edge_s = [ix >= EDGE_START for ix in idx_s]
        b16_s = [((ix % LANE) // 16) * 16 for ix in idx_s]
        o16_s = [ix % 16 for ix in idx_s]
        ct_eff_s = [jnp.where(edge_s[r], 0, ct_s[r]) for r in range(8)]

        for r in range(8):
            hb = r % 2
            if last_desc[hb] is not None:
                last_desc[hb].wait()
            ct = ct_eff_s[r]
            for r2 in range(8):
                contrib = jnp.where(
                    (ct_s[r2] == ct) & jnp.logical_not(edge_s[r2]),
                    jnp.float32(20.0), jnp.float32(0.0))
                pbs[hb][r2, pl.ds(b16_s[r2], 16)] = jnp.where(
                    lane == o16_s[r2], contrib, jnp.float32(0.0))
            d = pltpu.make_async_copy(
                pbs[hb],
                out_ref.at[pl.ds(rows0, 8), pl.ds(ct * LANE, LANE)],
                sems[hb])
            d.start()
            last_desc[hb] = d

    last_desc[0].wait()
    last_desc[1].wait()


def kernel(input_ids):
    B, S = input_ids.shape
    ids = input_ids.reshape(-1).astype(jnp.int32)
    dense = _tc_fill(ids)
    ref = jax.new_ref(dense)
    _sc_patch(ids, ref)
    return ref[...].reshape(B, S, VOCAB_SIZE)
